# baseline (speedup 1.0000x reference)
import functools

import jax
import jax.numpy as jnp
from jax import lax
from jax.experimental import pallas as pl
from jax.experimental.pallas import tpu as pltpu
from jax.experimental.pallas import tpu_sc as plsc

DIM = 128
MARGIN = 1.0
LANES = 16


def _trans_e_body(
    pos_hbm,
    neg_hbm,
    emb_hbm,
    out_hbm,
    idx_p,
    idx_n,
    rows_p,
    rows_n,
    out_v,
    sem_ip,
    sem_in,
    sem_gp,
    sem_gn,
):
    cp_p = pltpu.make_async_copy(pos_hbm, idx_p, sem_ip)
    cp_n = pltpu.make_async_copy(neg_hbm, idx_n, sem_in)
    cp_p.start()
    cp_n.start()
    g_p = pltpu.make_async_copy(emb_hbm.at[idx_p], rows_p, sem_gp)
    g_n = pltpu.make_async_copy(emb_hbm.at[idx_n], rows_n, sem_gn)
    cp_p.wait()
    g_p.start()
    cp_n.wait()
    g_n.start()

    g_p.wait()
    acc = jnp.zeros((LANES,), jnp.float32)
    for j in range(DIM // LANES):
        s = pl.ds(j * LANES, LANES)
        acc = acc + jnp.abs(rows_p[0, s] + rows_p[1, s] - rows_p[2, s])
    g_n.wait()
    for j in range(DIM // LANES):
        s = pl.ds(j * LANES, LANES)
        acc = acc - jnp.abs(rows_n[0, s] + rows_n[1, s] - rows_n[2, s])

    lanes = lax.iota(jnp.int32, LANES)
    for shift in (8, 4, 2, 1):
        perm = lax.rem(lanes + shift, LANES)
        acc = acc + acc.at[perm].get(mode="promise_in_bounds")
    out_v[...] = jnp.maximum(acc + MARGIN, 0.0)
    pltpu.sync_copy(out_v.at[pl.ds(0, 1)], out_hbm)


@jax.jit
def _trans_e_loss(pos_idx, neg_idx, embeddings):
    mesh = plsc.VectorSubcoreMesh(
        core_axis_name="c", subcore_axis_name="s", num_cores=1, num_subcores=1
    )
    k = functools.partial(
        pl.kernel,
        out_type=jax.ShapeDtypeStruct((1,), jnp.float32),
        mesh=mesh,
        scratch_types=[
            pltpu.VMEM((3,), jnp.int32),
            pltpu.VMEM((3,), jnp.int32),
            pltpu.VMEM((3, DIM), jnp.float32),
            pltpu.VMEM((3, DIM), jnp.float32),
            pltpu.VMEM((LANES,), jnp.float32),
            pltpu.SemaphoreType.DMA,
            pltpu.SemaphoreType.DMA,
            pltpu.SemaphoreType.DMA,
            pltpu.SemaphoreType.DMA,
        ],
    )(_trans_e_body)
    return jnp.reshape(k(pos_idx, neg_idx, embeddings), ())


def kernel(pos_exmpl, neg_exmpl, embeddings):
    return _trans_e_loss(
        pos_exmpl.astype(jnp.int32), neg_exmpl.astype(jnp.int32), embeddings
    )


# speedup vs baseline: 1.0096x; 1.0096x over previous
import functools

import jax
import jax.numpy as jnp
from jax import lax
from jax.experimental import pallas as pl
from jax.experimental.pallas import tpu as pltpu
from jax.experimental.pallas import tpu_sc as plsc

DIM = 128
MARGIN = 1.0
LANES = 16


def _trans_e_body(
    pos_hbm,
    neg_hbm,
    emb_hbm,
    out_hbm,
    idx_p,
    idx_n,
    rows_p,
    rows_n,
    out_v,
    sem_ip,
    sem_in,
    sem_gp,
    sem_gn,
):
    cp_p = pltpu.make_async_copy(pos_hbm, idx_p, sem_ip)
    cp_n = pltpu.make_async_copy(neg_hbm, idx_n, sem_in)
    cp_p.start()
    cp_n.start()
    g_p = pltpu.make_async_copy(emb_hbm.at[idx_p], rows_p, sem_gp)
    g_n = pltpu.make_async_copy(emb_hbm.at[idx_n], rows_n, sem_gn)
    cp_p.wait()
    g_p.start()
    cp_n.wait()
    g_n.start()

    g_p.wait()
    acc = jnp.zeros((LANES,), jnp.float32)
    for j in range(DIM // LANES):
        s = pl.ds(j * LANES, LANES)
        acc = acc + jnp.abs(rows_p[0, s] + rows_p[1, s] - rows_p[2, s])
    g_n.wait()
    for j in range(DIM // LANES):
        s = pl.ds(j * LANES, LANES)
        acc = acc - jnp.abs(rows_n[0, s] + rows_n[1, s] - rows_n[2, s])

    lanes = lax.iota(jnp.int32, LANES)
    for shift in (8, 4, 2, 1):
        perm = lax.rem(lanes + shift, LANES)
        acc = acc + acc.at[perm].get(mode="promise_in_bounds")
    out_v[...] = jnp.maximum(acc + MARGIN, 0.0)
    pltpu.sync_copy(out_v.at[pl.ds(0, 1)], out_hbm)


@jax.jit
def _trans_e_loss(pos_idx, neg_idx, embeddings):
    mesh = plsc.VectorSubcoreMesh(
        core_axis_name="c", subcore_axis_name="s", num_cores=1, num_subcores=1
    )
    k = functools.partial(
        pl.kernel,
        out_type=jax.ShapeDtypeStruct((1,), jnp.float32),
        mesh=mesh,
        scratch_types=[
            pltpu.VMEM((3,), jnp.int32),
            pltpu.VMEM((3,), jnp.int32),
            pltpu.VMEM((3, DIM), jnp.float32),
            pltpu.VMEM((3, DIM), jnp.float32),
            pltpu.VMEM((LANES,), jnp.float32),
            pltpu.SemaphoreType.DMA,
            pltpu.SemaphoreType.DMA,
            pltpu.SemaphoreType.DMA,
            pltpu.SemaphoreType.DMA,
        ],
        compiler_params=pltpu.CompilerParams(skip_device_barrier=True),
    )(_trans_e_body)
    return jnp.reshape(k(pos_idx, neg_idx, embeddings), ())


def kernel(pos_exmpl, neg_exmpl, embeddings):
    return _trans_e_loss(
        pos_exmpl.astype(jnp.int32), neg_exmpl.astype(jnp.int32), embeddings
    )
